# baseline (speedup 1.0000x reference)
import functools

import jax
import jax.numpy as jnp
from jax import lax
from jax.experimental import pallas as pl
from jax.experimental.pallas import tpu as pltpu
from jax.experimental.pallas import tpu_sc as plsc

VOCAB = 100000
D = 48
B = 16384
NF = 4
NC, NS = 2, 16
NW = NC * NS
BPW = B // NW
CH = 128
NCH = BPW // CH

_MESH = plsc.VectorSubcoreMesh(core_axis_name="c", subcore_axis_name="s")


def _body(i0, i1, i2, i3, w0, w1, w2, w3, out,
          idx_v, f0, f1, f2, f3, sem):
    wid = lax.axis_index("s") * NC + lax.axis_index("c")
    base = wid * BPW
    idx_refs = (i0, i1, i2, i3)
    tabs = (w0, w1, w2, w3)
    feats = (f0, f1, f2, f3)

    for f in range(NF):
        for c in range(NCH):
            pltpu.sync_copy(idx_refs[f].at[pl.ds(base + c * CH, CH)],
                            idx_v.at[f, c])

    copies = []
    for f in range(NF):
        for c in range(NCH):
            copies.append(pltpu.async_copy(
                tabs[f].at[idx_v.at[f, c]],
                feats[f].at[pl.ds(c * CH, CH)],
                sem))
    for cp in copies:
        cp.wait()

    for f in range(NF):
        pltpu.sync_copy(feats[f],
                        out.at[pl.ds(base, BPW), pl.ds(f * D, D)])


@jax.jit
def _lookup(i0, i1, i2, i3, w0, w1, w2, w3):
    return pl.kernel(
        _body,
        out_type=jax.ShapeDtypeStruct((B, NF * D), jnp.float32),
        mesh=_MESH,
        scratch_types=[
            pltpu.VMEM((NF, NCH, CH), jnp.int32),
            pltpu.VMEM((BPW, D), jnp.float32),
            pltpu.VMEM((BPW, D), jnp.float32),
            pltpu.VMEM((BPW, D), jnp.float32),
            pltpu.VMEM((BPW, D), jnp.float32),
            pltpu.SemaphoreType.DMA,
        ],
        compiler_params=pltpu.CompilerParams(use_tc_tiling_on_sc=False),
    )(i0, i1, i2, i3, w0, w1, w2, w3)


def kernel(idx_course_id, idx_instructor, idx_category, idx_school,
           W_course_id, W_instructor, W_category, W_school):
    return _lookup(idx_course_id, idx_instructor, idx_category, idx_school,
                   W_course_id, W_instructor, W_category, W_school)
